# baseline (speedup 1.0000x reference)
import jax
import jax.numpy as jnp
from jax.experimental import pallas as pl
from jax.experimental.pallas import tpu as pltpu


def _pair_kernel(ut_ref, it_ref, w1u_ref, w1v_ref, b1_ref, w2_ref, b2_ref,
                 wf_ref, bf_ref, o_ref):
    au = jnp.dot(w1u_ref[...], ut_ref[...], preferred_element_type=jnp.float32)
    bv = (jnp.dot(w1v_ref[...], it_ref[...], preferred_element_type=jnp.float32)
          + b1_ref[...])
    ub = au.shape[1]
    for u in range(ub):
        h1 = jnp.maximum(au[:, u:u + 1] + bv, 0.0)
        h2 = jnp.maximum(
            jnp.dot(w2_ref[...], h1, preferred_element_type=jnp.float32)
            + b2_ref[...], 0.0)
        logit = jnp.dot(wf_ref[...], h2,
                        preferred_element_type=jnp.float32) + bf_ref[0, 0]
        o_ref[u:u + 1, :] = jax.nn.sigmoid(logit)


def _lookup_kernel(uidx_ref, iidx_ref, t_ref, o_ref):
    uid = uidx_ref[...]
    iid = iidx_ref[...]
    nu, ni = t_ref.shape
    tb = uid.shape[1]

    ohi = (jax.lax.broadcasted_iota(jnp.int16, (ni, tb), 0)
           == iid.astype(jnp.int16)).astype(jnp.bfloat16)
    c = jnp.dot(t_ref[...], ohi,
                preferred_element_type=jnp.float32)

    d = c
    bit = nu >> 1
    while d.shape[0] > 8:
        half = d.shape[0] // 2
        d = jnp.where((uid & bit) != 0, d[half:, :], d[:half, :])
        bit >>= 1
    o_ref[...] = jnp.take_along_axis(d, uid & 7, axis=0)


def _forward(user_input, item_input, params, *, tb=16384, ub=128):
    user_table, item_table, w1, b1, w2, b2, wf, bf = params
    E = user_table.shape[1]
    Nu = user_table.shape[0]
    Ni = item_table.shape[0]
    H1 = w1.shape[1]
    H2 = w2.shape[1]

    B = user_input.shape[0]
    tb = min(tb, ((B + 127) // 128) * 128)
    nb = pl.cdiv(B, tb)
    B_pad = nb * tb
    if B_pad != B:
        pad = B_pad - B
        user_input = jnp.concatenate(
            [user_input, jnp.zeros((pad,), user_input.dtype)])
        item_input = jnp.concatenate(
            [item_input, jnp.zeros((pad,), item_input.dtype)])

    uidx = user_input.astype(jnp.int32).reshape(1, B_pad)
    iidx = item_input.astype(jnp.int32).reshape(1, B_pad)

    ut = user_table.T
    it = item_table.T
    w1u_t = w1[:E].T
    w1v_t = w1[E:].T
    b1_c = b1.reshape(H1, 1)
    w2_t = w2.T
    b2_c = b2.reshape(H2, 1)
    wf_t = wf.T
    bf_s = bf.reshape(1, 1)

    nub = pl.cdiv(Nu, ub)
    table = pl.pallas_call(
        _pair_kernel,
        out_shape=jax.ShapeDtypeStruct((Nu, Ni), jnp.float32),
        grid_spec=pltpu.PrefetchScalarGridSpec(
            num_scalar_prefetch=0,
            grid=(nub,),
            in_specs=[
                pl.BlockSpec((E, ub), lambda b: (0, b)),
                pl.BlockSpec((E, Ni), lambda b: (0, 0)),
                pl.BlockSpec((H1, E), lambda b: (0, 0)),
                pl.BlockSpec((H1, E), lambda b: (0, 0)),
                pl.BlockSpec((H1, 1), lambda b: (0, 0)),
                pl.BlockSpec((H2, H1), lambda b: (0, 0)),
                pl.BlockSpec((H2, 1), lambda b: (0, 0)),
                pl.BlockSpec((1, H2), lambda b: (0, 0)),
                pl.BlockSpec(memory_space=pltpu.MemorySpace.SMEM),
            ],
            out_specs=pl.BlockSpec((ub, Ni), lambda b: (b, 0)),
        ),
        compiler_params=pltpu.CompilerParams(
            dimension_semantics=("parallel",)),
    )(ut, it, w1u_t, w1v_t, b1_c, w2_t, b2_c, wf_t, bf_s)

    table_bf = table.astype(jnp.bfloat16)

    out = pl.pallas_call(
        _lookup_kernel,
        out_shape=jax.ShapeDtypeStruct((1, B_pad), jnp.float32),
        grid_spec=pltpu.PrefetchScalarGridSpec(
            num_scalar_prefetch=0,
            grid=(nb,),
            in_specs=[
                pl.BlockSpec((1, tb), lambda b: (0, b)),
                pl.BlockSpec((1, tb), lambda b: (0, b)),
                pl.BlockSpec((Nu, Ni), lambda b: (0, 0)),
            ],
            out_specs=pl.BlockSpec((1, tb), lambda b: (0, b)),
        ),
        compiler_params=pltpu.CompilerParams(
            dimension_semantics=("parallel",)),
    )(uidx, iidx, table_bf)

    return out.reshape(-1)[:B].reshape(B, 1)


def kernel(user_input, item_input, user_table, item_table, w1, b1, w2, b2,
           wf, bf):
    params = (user_table, item_table, w1, b1, w2, b2, wf, bf)
    return _forward(user_input, item_input, params)


# speedup vs baseline: 1.6732x; 1.6732x over previous
import jax
import jax.numpy as jnp
from jax.experimental import pallas as pl
from jax.experimental.pallas import tpu as pltpu


def _pair_kernel(ut_ref, it_ref, w1u_ref, w1v_ref, b1_ref, w2_ref, b2_ref,
                 wf_ref, bf_ref, o_ref):
    au = jnp.dot(w1u_ref[...], ut_ref[...], preferred_element_type=jnp.float32)
    bv = (jnp.dot(w1v_ref[...], it_ref[...], preferred_element_type=jnp.float32)
          + b1_ref[...])
    ub = au.shape[1]
    ni = bv.shape[1]
    sb = 32
    for g in range(ub // sb):
        h1 = jnp.concatenate(
            [jnp.maximum(au[:, u:u + 1] + bv, 0.0)
             for u in range(g * sb, (g + 1) * sb)],
            axis=1)
        h2 = jnp.maximum(
            jnp.dot(w2_ref[...], h1, preferred_element_type=jnp.float32)
            + b2_ref[...], 0.0)
        logit = jnp.dot(wf_ref[...], h2,
                        preferred_element_type=jnp.float32) + bf_ref[0, 0]
        o_ref[0:1, g * sb * ni:(g + 1) * sb * ni] = jax.nn.sigmoid(logit)


def _lookup_kernel(uidx_ref, iidx_ref, t_ref, o_ref):
    uid = uidx_ref[...]
    iid = iidx_ref[...]
    nu, ni = t_ref.shape
    tb = uid.shape[1]

    ohi = (jax.lax.broadcasted_iota(jnp.int32, (ni, tb), 0)
           == iid).astype(jnp.float32)
    c = jnp.dot(t_ref[...], ohi,
                preferred_element_type=jnp.float32)

    d = c
    bit = nu >> 1
    while d.shape[0] > 8:
        half = d.shape[0] // 2
        d = jnp.where((uid & bit) != 0, d[half:, :], d[:half, :])
        bit >>= 1
    o_ref[...] = jnp.take_along_axis(d, uid & 7, axis=0)


def _forward(user_input, item_input, params, *, tb=16384, ub=128):
    user_table, item_table, w1, b1, w2, b2, wf, bf = params
    E = user_table.shape[1]
    Nu = user_table.shape[0]
    Ni = item_table.shape[0]
    H1 = w1.shape[1]
    H2 = w2.shape[1]

    B = user_input.shape[0]
    tb = min(tb, ((B + 127) // 128) * 128)
    nb = pl.cdiv(B, tb)
    B_pad = nb * tb
    if B_pad != B:
        pad = B_pad - B
        user_input = jnp.concatenate(
            [user_input, jnp.zeros((pad,), user_input.dtype)])
        item_input = jnp.concatenate(
            [item_input, jnp.zeros((pad,), item_input.dtype)])

    uidx = user_input.astype(jnp.int32).reshape(1, B_pad)
    iidx = item_input.astype(jnp.int32).reshape(1, B_pad)

    ut = user_table.T
    it = item_table.T
    w1u_t = w1[:E].T
    w1v_t = w1[E:].T
    b1_c = b1.reshape(H1, 1)
    w2_t = w2.T
    b2_c = b2.reshape(H2, 1)
    wf_t = wf.T
    bf_s = bf.reshape(1, 1)

    nub = pl.cdiv(Nu, ub)
    table_row = pl.pallas_call(
        _pair_kernel,
        out_shape=jax.ShapeDtypeStruct((1, Nu * Ni), jnp.float32),
        grid_spec=pltpu.PrefetchScalarGridSpec(
            num_scalar_prefetch=0,
            grid=(nub,),
            in_specs=[
                pl.BlockSpec((E, ub), lambda b: (0, b)),
                pl.BlockSpec((E, Ni), lambda b: (0, 0)),
                pl.BlockSpec((H1, E), lambda b: (0, 0)),
                pl.BlockSpec((H1, E), lambda b: (0, 0)),
                pl.BlockSpec((H1, 1), lambda b: (0, 0)),
                pl.BlockSpec((H2, H1), lambda b: (0, 0)),
                pl.BlockSpec((H2, 1), lambda b: (0, 0)),
                pl.BlockSpec((1, H2), lambda b: (0, 0)),
                pl.BlockSpec(memory_space=pltpu.MemorySpace.SMEM),
            ],
            out_specs=pl.BlockSpec((1, ub * Ni), lambda b: (0, b)),
        ),
        compiler_params=pltpu.CompilerParams(
            dimension_semantics=("parallel",)),
    )(ut, it, w1u_t, w1v_t, b1_c, w2_t, b2_c, wf_t, bf_s)

    table = table_row.reshape(Nu, Ni)

    out = pl.pallas_call(
        _lookup_kernel,
        out_shape=jax.ShapeDtypeStruct((1, B_pad), jnp.float32),
        grid_spec=pltpu.PrefetchScalarGridSpec(
            num_scalar_prefetch=0,
            grid=(nb,),
            in_specs=[
                pl.BlockSpec((1, tb), lambda b: (0, b)),
                pl.BlockSpec((1, tb), lambda b: (0, b)),
                pl.BlockSpec((Nu, Ni), lambda b: (0, 0)),
            ],
            out_specs=pl.BlockSpec((1, tb), lambda b: (0, b)),
        ),
        compiler_params=pltpu.CompilerParams(
            dimension_semantics=("parallel",)),
    )(uidx, iidx, table)

    return out.reshape(-1)[:B].reshape(B, 1)


def kernel(user_input, item_input, user_table, item_table, w1, b1, w2, b2,
           wf, bf):
    params = (user_table, item_table, w1, b1, w2, b2, wf, bf)
    return _forward(user_input, item_input, params)
